# baseline (speedup 1.0000x reference)
import functools

import jax
import jax.numpy as jnp
from jax import lax
from jax.experimental import pallas as pl
from jax.experimental.pallas import tpu as pltpu
from jax.experimental.pallas import tpu_sc as plsc

N_ORIG = 1000000
N_NEW = 1024
D = 64
B = 16384
L = 50

NC = 2
NS = 16
NW = NC * NS

TOKENS = B * L
TPW = TOKENS // NW
CHUNK = 512
NCHUNK = TPW // CHUNK
SUB = 128
NSUB = CHUNK // SUB


def _body(ids_hbm, orig_hbm, new_hbm, out_hbm,
          idx0, idx1, oidx0, oidx1, rows0, rows1, patchbuf,
          gsem0, gsem1, wsem0, wsem1, psem):
    cid = lax.axis_index("c")
    sid = lax.axis_index("s")
    wid = sid * NC + cid
    base = wid * TPW

    idxb = (idx0, idx1)
    oidxb = (oidx0, oidx1)
    rowsb = (rows0, rows1)
    gsems = (gsem0, gsem1)
    wsems = (wsem0, wsem1)

    def load_fire(c, j):
        cb = base + c * CHUNK
        pltpu.sync_copy(ids_hbm.at[pl.ds(cb, CHUNK)], idxb[j])

        @pl.loop(0, CHUNK // 16)
        def _grp(g):
            v = idxb[j][pl.ds(g * 16, 16)]
            oidxb[j][pl.ds(g * 16, 16)] = jnp.where(v >= N_ORIG, 0, v)

        for s in range(NSUB):
            pltpu.async_copy(
                orig_hbm.at[oidxb[j].at[pl.ds(s * SUB, SUB)]],
                rowsb[j].at[pl.ds(s * SUB, SUB)],
                gsems[j],
            )

    def drain_gathers(j):
        for s in range(NSUB):
            pltpu.make_async_copy(
                orig_hbm.at[oidxb[j].at[pl.ds(s * SUB, SUB)]],
                rowsb[j].at[pl.ds(s * SUB, SUB)],
                gsems[j],
            ).wait()

    def fire_writeout(c, j):
        cb = base + c * CHUNK
        pltpu.async_copy(rowsb[j], out_hbm.at[pl.ds(cb, CHUNK)], wsems[j])

    def drain_writeout(c, j):
        cb = base + c * CHUNK
        pltpu.make_async_copy(
            rowsb[j], out_hbm.at[pl.ds(cb, CHUNK)], wsems[j]).wait()

    def patch(c, j):
        cb = base + c * CHUNK

        @pl.loop(0, CHUNK // 16)
        def _fix(g):
            v = idxb[j][pl.ds(g * 16, 16)]
            m = v >= N_ORIG
            cnt = plsc.all_reduce_population_count(m)[0]

            @pl.when(cnt > 0)
            def _():
                lane = lax.broadcasted_iota(jnp.int32, (16,), 0)
                nv = jnp.where(m, v - N_ORIG, 0)
                pltpu.async_copy(new_hbm.at[nv], patchbuf, psem).wait()
                gdst = jnp.where(m, cb + g * 16 + lane, TOKENS + lane)
                pltpu.async_copy(patchbuf, out_hbm.at[gdst], psem).wait()

    load_fire(0, 0)

    @pl.loop(0, NCHUNK // 2)
    def _pair(p):
        for jj in range(2):
            c = 2 * p + jj
            prev = c - 1
            nxt = c + 1
            oj = 1 - jj

            if jj == 0:
                @pl.when(p > 0)
                def _():
                    drain_writeout(prev, oj)
                    patch(prev, oj)
                load_fire(nxt, oj)
            else:
                drain_writeout(prev, oj)
                patch(prev, oj)

                @pl.when(p < NCHUNK // 2 - 1)
                def _():
                    load_fire(nxt, oj)

            drain_gathers(jj)
            fire_writeout(c, jj)

    drain_writeout(NCHUNK - 1, 1)
    patch(NCHUNK - 1, 1)


@functools.partial(jax.jit, static_argnames=())
def kernel(input_ids, original_weight, new_weight):
    ids_flat = input_ids.reshape(TOKENS)
    mesh = plsc.VectorSubcoreMesh(
        core_axis_name="c", subcore_axis_name="s",
        num_cores=NC, num_subcores=NS,
    )
    out = pl.kernel(
        _body,
        out_type=jax.ShapeDtypeStruct((TOKENS + 16, D), jnp.float32),
        mesh=mesh,
        compiler_params=pltpu.CompilerParams(
            use_tc_tiling_on_sc=False, needs_layout_passes=False),
        scratch_types=[
            pltpu.VMEM((CHUNK,), jnp.int32),
            pltpu.VMEM((CHUNK,), jnp.int32),
            pltpu.VMEM((CHUNK,), jnp.int32),
            pltpu.VMEM((CHUNK,), jnp.int32),
            pltpu.VMEM((CHUNK, D), jnp.float32),
            pltpu.VMEM((CHUNK, D), jnp.float32),
            pltpu.VMEM((16, D), jnp.float32),
            pltpu.SemaphoreType.DMA,
            pltpu.SemaphoreType.DMA,
            pltpu.SemaphoreType.DMA,
            pltpu.SemaphoreType.DMA,
            pltpu.SemaphoreType.DMA,
        ],
    )(ids_flat, original_weight, new_weight)
    return out[:TOKENS].reshape(B, L, D)


# speedup vs baseline: 1.0781x; 1.0781x over previous
import functools

import jax
import jax.numpy as jnp
from jax import lax
from jax.experimental import pallas as pl
from jax.experimental.pallas import tpu as pltpu
from jax.experimental.pallas import tpu_sc as plsc

N_ORIG = 1000000
N_NEW = 1024
D = 64
B = 16384
L = 50

NC = 2
NS = 16
NW = NC * NS

TOKENS = B * L
BB = B // 128
UNITS = L * BB
UPW = UNITS // NW


def _body(ids_hbm, orig_hbm, new_hbm, out_hbm,
          idx0, idx1, oidx0, oidx1, rows0, rows1, rowst0, rowst1,
          gsem0, gsem1, wsem0, wsem1, psem):
    cid = lax.axis_index("c")
    sid = lax.axis_index("s")
    wid = sid * NC + cid
    ubase = wid * UPW

    idxb = (idx0, idx1)
    oidxb = (oidx0, oidx1)
    rowsb = (rows0, rows1)
    rowstb = (rowst0, rowst1)
    gsems = (gsem0, gsem1)
    wsems = (wsem0, wsem1)

    def unit_lb(u):
        return u // BB, u % BB

    def load_fire(u, j):
        l, bblk = unit_lb(u)
        pltpu.sync_copy(ids_hbm.at[pl.ds(l * B + bblk * 128, 128)], idxb[j])

        @pl.loop(0, 8)
        def _grp(g):
            v = idxb[j][pl.ds(g * 16, 16)]
            oidxb[j][pl.ds(g * 16, 16)] = jnp.where(v >= N_ORIG, 0, v)

        pltpu.async_copy(orig_hbm.at[oidxb[j]], rowsb[j], gsems[j])

    def drain_gather(j):
        pltpu.make_async_copy(
            orig_hbm.at[oidxb[j]], rowsb[j], gsems[j]).wait()

    def patch(u, j):
        l, bblk = unit_lb(u)

        @pl.loop(0, 8)
        def _fix(g):
            v = idxb[j][pl.ds(g * 16, 16)]
            m = v >= N_ORIG
            cnt = plsc.all_reduce_population_count(m)[0]

            @pl.when(cnt > 0)
            def _():
                for t in range(16):
                    vid = v[t]

                    @pl.when(vid >= N_ORIG)
                    def _():
                        pltpu.async_copy(
                            new_hbm.at[pl.ds(vid - N_ORIG, 1)],
                            rowsb[j].at[pl.ds(g * 16 + t, 1)],
                            psem,
                        ).wait()

    def transpose_unit(j):
        lane = lax.broadcasted_iota(jnp.int32, (16,), 0)

        @pl.loop(0, D)
        def _d(d):
            dvec = jnp.full((16,), 0, jnp.int32) + d

            @pl.loop(0, 8)
            def _g(g):
                bvec = g * 16 + lane
                vals = plsc.load_gather(rowsb[j], [bvec, dvec])
                rowstb[j][d, pl.ds(g * 16, 16)] = vals

    def fire_write(u, j):
        l, bblk = unit_lb(u)
        for dblk in range(8):
            pltpu.async_copy(
                rowstb[j].at[pl.ds(dblk * 8, 8)],
                out_hbm.at[l, dblk, bblk],
                wsems[j],
            )

    def drain_write(u, j):
        l, bblk = unit_lb(u)
        for dblk in range(8):
            pltpu.make_async_copy(
                rowstb[j].at[pl.ds(dblk * 8, 8)],
                out_hbm.at[l, dblk, bblk],
                wsems[j],
            ).wait()

    load_fire(ubase, 0)

    @pl.loop(0, UPW // 2)
    def _pair(p):
        for jj in range(2):
            u = ubase + 2 * p + jj
            oj = 1 - jj

            if jj == 0:
                @pl.when(p > 0)
                def _():
                    drain_write(u - 1, oj)
                load_fire(u + 1, oj)
            else:
                drain_write(u - 1, oj)

                @pl.when(p < UPW // 2 - 1)
                def _():
                    load_fire(u + 1, oj)

            drain_gather(jj)
            patch(u, jj)
            transpose_unit(jj)
            fire_write(u, jj)

    drain_write(ubase + UPW - 1, 1)


@functools.partial(jax.jit, static_argnames=())
def kernel(input_ids, original_weight, new_weight):
    ids_lmajor = input_ids.T.reshape(TOKENS)
    mesh = plsc.VectorSubcoreMesh(
        core_axis_name="c", subcore_axis_name="s",
        num_cores=NC, num_subcores=NS,
    )
    out5d = pl.kernel(
        _body,
        out_type=jax.ShapeDtypeStruct((L, 8, BB, 8, 128), jnp.float32),
        mesh=mesh,
        compiler_params=pltpu.CompilerParams(
            use_tc_tiling_on_sc=False, needs_layout_passes=False),
        scratch_types=[
            pltpu.VMEM((128,), jnp.int32),
            pltpu.VMEM((128,), jnp.int32),
            pltpu.VMEM((128,), jnp.int32),
            pltpu.VMEM((128,), jnp.int32),
            pltpu.VMEM((128, D), jnp.float32),
            pltpu.VMEM((128, D), jnp.float32),
            pltpu.VMEM((D, 128), jnp.float32),
            pltpu.VMEM((D, 128), jnp.float32),
            pltpu.SemaphoreType.DMA,
            pltpu.SemaphoreType.DMA,
            pltpu.SemaphoreType.DMA,
            pltpu.SemaphoreType.DMA,
            pltpu.SemaphoreType.DMA,
        ],
    )(ids_lmajor, original_weight, new_weight)
    return out5d.transpose(2, 4, 0, 1, 3).reshape(B, L, D)


# speedup vs baseline: 1.1299x; 1.0481x over previous
import functools

import jax
import jax.numpy as jnp
from jax import lax
from jax.experimental import pallas as pl
from jax.experimental.pallas import tpu as pltpu
from jax.experimental.pallas import tpu_sc as plsc

N_ORIG = 1000000
N_NEW = 1024
D = 64
B = 16384
L = 50

NC = 2
NS = 16
NW = NC * NS

TOKENS = B * L
BB = B // 128
UNITS = L * BB
UPW = UNITS // NW


def _body(ids_hbm, orig_hbm, new_hbm, out_hbm,
          idx0, idx1, oidx0, oidx1, rows0, rows1, rowst0, rowst1,
          gsem0, gsem1, wsem0, wsem1, psem):
    cid = lax.axis_index("c")
    sid = lax.axis_index("s")
    wid = sid * NC + cid
    ubase = wid * UPW

    idxb = (idx0, idx1)
    oidxb = (oidx0, oidx1)
    rowsb = (rows0, rows1)
    rowstb = (rowst0, rowst1)
    gsems = (gsem0, gsem1)
    wsems = (wsem0, wsem1)

    def unit_lb(u):
        return u // BB, u % BB

    def load_fire(u, j):
        l, bblk = unit_lb(u)
        pltpu.sync_copy(ids_hbm.at[pl.ds(l * B + bblk * 128, 128)], idxb[j])

        @pl.loop(0, 8)
        def _grp(g):
            v = idxb[j][pl.ds(g * 16, 16)]
            oidxb[j][pl.ds(g * 16, 16)] = jnp.where(v >= N_ORIG, 0, v)

        pltpu.async_copy(orig_hbm.at[oidxb[j]], rowsb[j], gsems[j])

    def drain_gather(j):
        pltpu.make_async_copy(
            orig_hbm.at[oidxb[j]], rowsb[j], gsems[j]).wait()

    def patch(u, j):
        l, bblk = unit_lb(u)

        @pl.loop(0, 8)
        def _fix(g):
            v = idxb[j][pl.ds(g * 16, 16)]
            m = v >= N_ORIG
            cnt = plsc.all_reduce_population_count(m)[0]

            @pl.when(cnt > 0)
            def _():
                for t in range(16):
                    vid = v[t]

                    @pl.when(vid >= N_ORIG)
                    def _():
                        pltpu.async_copy(
                            new_hbm.at[pl.ds(vid - N_ORIG, 1)],
                            rowsb[j].at[pl.ds(g * 16 + t, 1)],
                            psem,
                        ).wait()

    def transpose_unit(j):
        lane = lax.broadcasted_iota(jnp.int32, (16,), 0)

        @pl.loop(0, D, unroll=8)
        def _d(d):
            dvec = jnp.full((16,), 0, jnp.int32) + d
            for g in range(8):
                bvec = g * 16 + lane
                vals = plsc.load_gather(rowsb[j], [bvec, dvec])
                rowstb[j][d, pl.ds(g * 16, 16)] = vals

    def fire_write(u, j):
        l, bblk = unit_lb(u)
        for dblk in range(8):
            pltpu.async_copy(
                rowstb[j].at[pl.ds(dblk * 8, 8)],
                out_hbm.at[l, dblk, bblk],
                wsems[j],
            )

    def drain_write(u, j):
        l, bblk = unit_lb(u)
        for dblk in range(8):
            pltpu.make_async_copy(
                rowstb[j].at[pl.ds(dblk * 8, 8)],
                out_hbm.at[l, dblk, bblk],
                wsems[j],
            ).wait()

    load_fire(ubase, 0)

    @pl.loop(0, UPW // 2)
    def _pair(p):
        for jj in range(2):
            u = ubase + 2 * p + jj
            oj = 1 - jj

            if jj == 0:
                @pl.when(p > 0)
                def _():
                    drain_write(u - 1, oj)
                load_fire(u + 1, oj)
            else:
                drain_write(u - 1, oj)

                @pl.when(p < UPW // 2 - 1)
                def _():
                    load_fire(u + 1, oj)

            drain_gather(jj)
            patch(u, jj)
            transpose_unit(jj)
            fire_write(u, jj)

    drain_write(ubase + UPW - 1, 1)


@functools.partial(jax.jit, static_argnames=())
def kernel(input_ids, original_weight, new_weight):
    ids_lmajor = input_ids.T.reshape(TOKENS)
    mesh = plsc.VectorSubcoreMesh(
        core_axis_name="c", subcore_axis_name="s",
        num_cores=NC, num_subcores=NS,
    )
    out5d = pl.kernel(
        _body,
        out_type=jax.ShapeDtypeStruct((L, 8, BB, 8, 128), jnp.float32),
        mesh=mesh,
        compiler_params=pltpu.CompilerParams(
            use_tc_tiling_on_sc=False, needs_layout_passes=False),
        scratch_types=[
            pltpu.VMEM((128,), jnp.int32),
            pltpu.VMEM((128,), jnp.int32),
            pltpu.VMEM((128,), jnp.int32),
            pltpu.VMEM((128,), jnp.int32),
            pltpu.VMEM((128, D), jnp.float32),
            pltpu.VMEM((128, D), jnp.float32),
            pltpu.VMEM((D, 128), jnp.float32),
            pltpu.VMEM((D, 128), jnp.float32),
            pltpu.SemaphoreType.DMA,
            pltpu.SemaphoreType.DMA,
            pltpu.SemaphoreType.DMA,
            pltpu.SemaphoreType.DMA,
            pltpu.SemaphoreType.DMA,
        ],
    )(ids_lmajor, original_weight, new_weight)
    return out5d.transpose(2, 4, 0, 1, 3).reshape(B, L, D)


# speedup vs baseline: 1.2866x; 1.1386x over previous
import functools

import jax
import jax.numpy as jnp
from jax import lax
from jax.experimental import pallas as pl
from jax.experimental.pallas import tpu as pltpu
from jax.experimental.pallas import tpu_sc as plsc

N_ORIG = 1000000
N_NEW = 1024
D = 64
B = 16384
L = 50

NC = 2
NS = 16
NW = NC * NS

TOKENS = B * L
BB = B // 128
UNITS = L * BB
UPW = UNITS // NW


def _body(ids_hbm, orig_hbm, new_hbm, out_hbm,
          idx0, idx1, oidx0, oidx1, rows0, rows1, rowst0, rowst1,
          gsem0, gsem1, wsem0, wsem1, psem):
    cid = lax.axis_index("c")
    sid = lax.axis_index("s")
    wid = sid * NC + cid
    ubase = wid * UPW

    idxb = (idx0, idx1)
    oidxb = (oidx0, oidx1)
    rowsb = (rows0, rows1)
    rowstb = (rowst0, rowst1)
    gsems = (gsem0, gsem1)
    wsems = (wsem0, wsem1)

    def unit_lb(u):
        return u // BB, u % BB

    def load_fire(u, j):
        l, bblk = unit_lb(u)
        pltpu.sync_copy(ids_hbm.at[pl.ds(l * B + bblk * 128, 128)], idxb[j])

        @pl.loop(0, 8)
        def _grp(g):
            v = idxb[j][pl.ds(g * 16, 16)]
            oidxb[j][pl.ds(g * 16, 16)] = jnp.where(v >= N_ORIG, 0, v)

        pltpu.async_copy(orig_hbm.at[oidxb[j]], rowsb[j], gsems[j])

    def drain_gather(j):
        pltpu.make_async_copy(
            orig_hbm.at[oidxb[j]], rowsb[j], gsems[j]).wait()

    def patch(u, j):
        l, bblk = unit_lb(u)

        @pl.loop(0, 8)
        def _fix(g):
            v = idxb[j][pl.ds(g * 16, 16)]
            m = v >= N_ORIG
            cnt = plsc.all_reduce_population_count(m)[0]

            @pl.when(cnt > 0)
            def _():
                for t in range(16):
                    vid = v[t]

                    @pl.when(vid >= N_ORIG)
                    def _():
                        pltpu.async_copy(
                            new_hbm.at[pl.ds(vid - N_ORIG, 1)],
                            rowsb[j].at[pl.ds(g * 16 + t, 1)],
                            psem,
                        ).wait()

    def transpose_unit(j):
        lane = lax.broadcasted_iota(jnp.int32, (16,), 0)

        @pl.loop(0, 128, unroll=8)
        def _t(t):
            tvec = jnp.full((16,), 0, jnp.int32) + t
            for q in range(4):
                vals = rowsb[j][t, pl.ds(q * 16, 16)]
                dvec = q * 16 + lane
                plsc.store_scatter(rowstb[j], [dvec, tvec], vals)

    def fire_write(u, j):
        l, bblk = unit_lb(u)
        for dblk in range(8):
            pltpu.async_copy(
                rowstb[j].at[pl.ds(dblk * 8, 8)],
                out_hbm.at[l, dblk, bblk],
                wsems[j],
            )

    def drain_write(u, j):
        l, bblk = unit_lb(u)
        for dblk in range(8):
            pltpu.make_async_copy(
                rowstb[j].at[pl.ds(dblk * 8, 8)],
                out_hbm.at[l, dblk, bblk],
                wsems[j],
            ).wait()

    load_fire(ubase, 0)

    @pl.loop(0, UPW // 2)
    def _pair(p):
        for jj in range(2):
            u = ubase + 2 * p + jj
            oj = 1 - jj

            if jj == 0:
                @pl.when(p > 0)
                def _():
                    drain_write(u - 1, oj)
                load_fire(u + 1, oj)
            else:
                drain_write(u - 1, oj)

                @pl.when(p < UPW // 2 - 1)
                def _():
                    load_fire(u + 1, oj)

            drain_gather(jj)
            patch(u, jj)
            transpose_unit(jj)
            fire_write(u, jj)

    drain_write(ubase + UPW - 1, 1)


@functools.partial(jax.jit, static_argnames=())
def kernel(input_ids, original_weight, new_weight):
    ids_lmajor = input_ids.T.reshape(TOKENS)
    mesh = plsc.VectorSubcoreMesh(
        core_axis_name="c", subcore_axis_name="s",
        num_cores=NC, num_subcores=NS,
    )
    out5d = pl.kernel(
        _body,
        out_type=jax.ShapeDtypeStruct((L, 8, BB, 8, 128), jnp.float32),
        mesh=mesh,
        compiler_params=pltpu.CompilerParams(
            use_tc_tiling_on_sc=False, needs_layout_passes=False),
        scratch_types=[
            pltpu.VMEM((128,), jnp.int32),
            pltpu.VMEM((128,), jnp.int32),
            pltpu.VMEM((128,), jnp.int32),
            pltpu.VMEM((128,), jnp.int32),
            pltpu.VMEM((128, D), jnp.float32),
            pltpu.VMEM((128, D), jnp.float32),
            pltpu.VMEM((D, 128), jnp.float32),
            pltpu.VMEM((D, 128), jnp.float32),
            pltpu.SemaphoreType.DMA,
            pltpu.SemaphoreType.DMA,
            pltpu.SemaphoreType.DMA,
            pltpu.SemaphoreType.DMA,
            pltpu.SemaphoreType.DMA,
        ],
    )(ids_lmajor, original_weight, new_weight)
    return out5d.transpose(2, 4, 0, 1, 3).reshape(B, L, D)


# speedup vs baseline: 1.4008x; 1.0888x over previous
import functools

import jax
import jax.numpy as jnp
from jax import lax
from jax.experimental import pallas as pl
from jax.experimental.pallas import tpu as pltpu
from jax.experimental.pallas import tpu_sc as plsc

N_ORIG = 1000000
N_NEW = 1024
D = 64
B = 16384
L = 50

NC = 2
NS = 16
NW = NC * NS

TOKENS = B * L
BB = B // 128
UNITS = L * BB
UPW = UNITS // NW
IPW = UPW * 128
NBUF = 4


def _body(ids_hbm, orig_hbm, new_hbm, out_hbm,
          idbuf, oidbuf, rows, rowst, gsems, wsems, psem):
    cid = lax.axis_index("c")
    sid = lax.axis_index("s")
    wid = sid * NC + cid
    ubase = wid * UPW

    pltpu.sync_copy(ids_hbm.at[pl.ds(ubase * 128, IPW)], idbuf)

    @pl.loop(0, IPW // 16)
    def _grp(g):
        v = idbuf[pl.ds(g * 16, 16)]
        oidbuf[pl.ds(g * 16, 16)] = jnp.where(v >= N_ORIG, 0, v)

    def fire_gather(k, j):
        pltpu.async_copy(
            orig_hbm.at[oidbuf.at[pl.ds(k * 128, 128)]],
            rows[j], gsems[j])

    def drain_gather(k, j):
        pltpu.make_async_copy(
            orig_hbm.at[oidbuf.at[pl.ds(k * 128, 128)]],
            rows[j], gsems[j]).wait()

    def patch(k, j):
        @pl.loop(0, 8)
        def _fix(g):
            v = idbuf[pl.ds(k * 128 + g * 16, 16)]
            m = v >= N_ORIG
            cnt = plsc.all_reduce_population_count(m)[0]

            @pl.when(cnt > 0)
            def _():
                for t in range(16):
                    vid = v[t]

                    @pl.when(vid >= N_ORIG)
                    def _():
                        pltpu.async_copy(
                            new_hbm.at[pl.ds(vid - N_ORIG, 1)],
                            rows[j].at[pl.ds(g * 16 + t, 1)],
                            psem,
                        ).wait()

    def transpose_unit(j):
        lane = lax.broadcasted_iota(jnp.int32, (16,), 0)

        @pl.loop(0, 128, unroll=8)
        def _t(t):
            tvec = jnp.full((16,), 0, jnp.int32) + t
            for q in range(4):
                vals = rows[j][t, pl.ds(q * 16, 16)]
                dvec = q * 16 + lane
                plsc.store_scatter(rowst[j], [dvec, tvec], vals)

    def fire_write(k, j):
        u = ubase + k
        l = u // BB
        bblk = u % BB
        for dblk in range(8):
            pltpu.async_copy(
                rowst[j].at[pl.ds(dblk * 8, 8)],
                out_hbm.at[l, dblk, bblk],
                wsems[j],
            )

    def drain_write(k, j):
        u = ubase + k
        l = u // BB
        bblk = u % BB
        for dblk in range(8):
            pltpu.make_async_copy(
                rowst[j].at[pl.ds(dblk * 8, 8)],
                out_hbm.at[l, dblk, bblk],
                wsems[j],
            ).wait()

    for j in range(NBUF):
        fire_gather(j, j)

    @pl.loop(0, UPW // NBUF)
    def _ring(p):
        for j in range(NBUF):
            k = p * NBUF + j
            drain_gather(k, j)
            patch(k, j)

            @pl.when(p > 0)
            def _():
                drain_write(k - NBUF, j)

            transpose_unit(j)
            fire_write(k, j)

            @pl.when(p < UPW // NBUF - 1)
            def _():
                fire_gather(k + NBUF, j)

    for j in range(NBUF):
        drain_write(UPW - NBUF + j, j)


@functools.partial(jax.jit, static_argnames=())
def kernel(input_ids, original_weight, new_weight):
    ids_lmajor = input_ids.T.reshape(TOKENS)
    mesh = plsc.VectorSubcoreMesh(
        core_axis_name="c", subcore_axis_name="s",
        num_cores=NC, num_subcores=NS,
    )
    out5d = pl.kernel(
        _body,
        out_type=jax.ShapeDtypeStruct((L, 8, BB, 8, 128), jnp.float32),
        mesh=mesh,
        compiler_params=pltpu.CompilerParams(
            use_tc_tiling_on_sc=False, needs_layout_passes=False),
        scratch_types=[
            pltpu.VMEM((IPW,), jnp.int32),
            pltpu.VMEM((IPW,), jnp.int32),
            [pltpu.VMEM((128, D), jnp.float32) for _ in range(NBUF)],
            [pltpu.VMEM((D, 128), jnp.float32) for _ in range(NBUF)],
            [pltpu.SemaphoreType.DMA for _ in range(NBUF)],
            [pltpu.SemaphoreType.DMA for _ in range(NBUF)],
            pltpu.SemaphoreType.DMA,
        ],
    )(ids_lmajor, original_weight, new_weight)
    return out5d.transpose(2, 4, 0, 1, 3).reshape(B, L, D)


# speedup vs baseline: 1.4017x; 1.0006x over previous
import functools

import jax
import jax.numpy as jnp
from jax import lax
from jax.experimental import pallas as pl
from jax.experimental.pallas import tpu as pltpu
from jax.experimental.pallas import tpu_sc as plsc

N_ORIG = 1000000
N_NEW = 1024
D = 64
B = 16384
L = 50

NC = 2
NS = 16
NW = NC * NS

TOKENS = B * L
BB = B // 128
UNITS = L * BB
UPW = UNITS // NW
IPW = UPW * 128
NBUF = 4


def _body(ids_hbm, orig_hbm, new_hbm, out_hbm,
          idbuf, oidbuf, rows, rowst, gsems, wsems, psem):
    cid = lax.axis_index("c")
    sid = lax.axis_index("s")
    wid = sid * NC + cid
    ubase = wid * UPW

    pltpu.sync_copy(ids_hbm.at[pl.ds(ubase * 128, IPW)], idbuf)

    @pl.loop(0, IPW // 16)
    def _grp(g):
        v = idbuf[pl.ds(g * 16, 16)]
        oidbuf[pl.ds(g * 16, 16)] = jnp.where(v >= N_ORIG, 0, v)

    def fire_gather(k, j):
        pltpu.async_copy(
            orig_hbm.at[oidbuf.at[pl.ds(k * 128, 128)]],
            rows[j], gsems[j])

    def drain_gather(k, j):
        pltpu.make_async_copy(
            orig_hbm.at[oidbuf.at[pl.ds(k * 128, 128)]],
            rows[j], gsems[j]).wait()

    def patch(k, j):
        @pl.loop(0, 8)
        def _fix(g):
            v = idbuf[pl.ds(k * 128 + g * 16, 16)]
            m = v >= N_ORIG
            cnt = plsc.all_reduce_population_count(m)[0]

            @pl.when(cnt > 0)
            def _():
                for t in range(16):
                    vid = v[t]

                    @pl.when(vid >= N_ORIG)
                    def _():
                        pltpu.async_copy(
                            new_hbm.at[pl.ds(vid - N_ORIG, 1)],
                            rows[j].at[pl.ds(g * 16 + t, 1)],
                            psem,
                        ).wait()

    def transpose_unit(j):
        lane = lax.broadcasted_iota(jnp.int32, (16,), 0)

        @pl.loop(0, 128, unroll=8)
        def _t(t):
            for q in range(4):
                vals = rows[j][t, pl.ds(q * 16, 16)]
                fidx = (q * 16 + lane) * 128 + t
                plsc.store_scatter(rowst[j], [fidx], vals)

    def fire_write(k, j):
        u = ubase + k
        l = u // BB
        bblk = u % BB
        for dblk in range(8):
            pltpu.async_copy(
                rowst[j].at[pl.ds(dblk * 1024, 1024)],
                out_hbm.at[l, dblk, bblk],
                wsems[j],
            )

    def drain_write(k, j):
        u = ubase + k
        l = u // BB
        bblk = u % BB
        for dblk in range(8):
            pltpu.make_async_copy(
                rowst[j].at[pl.ds(dblk * 1024, 1024)],
                out_hbm.at[l, dblk, bblk],
                wsems[j],
            ).wait()

    for j in range(NBUF):
        fire_gather(j, j)

    @pl.loop(0, UPW // NBUF)
    def _ring(p):
        for j in range(NBUF):
            k = p * NBUF + j
            drain_gather(k, j)
            patch(k, j)

            @pl.when(p > 0)
            def _():
                drain_write(k - NBUF, j)

            transpose_unit(j)
            fire_write(k, j)

            @pl.when(p < UPW // NBUF - 1)
            def _():
                fire_gather(k + NBUF, j)

    for j in range(NBUF):
        drain_write(UPW - NBUF + j, j)


@functools.partial(jax.jit, static_argnames=())
def kernel(input_ids, original_weight, new_weight):
    ids_lmajor = input_ids.T.reshape(TOKENS)
    mesh = plsc.VectorSubcoreMesh(
        core_axis_name="c", subcore_axis_name="s",
        num_cores=NC, num_subcores=NS,
    )
    out5d = pl.kernel(
        _body,
        out_type=jax.ShapeDtypeStruct((L, 8, BB, 1024), jnp.float32),
        mesh=mesh,
        compiler_params=pltpu.CompilerParams(
            use_tc_tiling_on_sc=False, needs_layout_passes=False),
        scratch_types=[
            pltpu.VMEM((IPW,), jnp.int32),
            pltpu.VMEM((IPW,), jnp.int32),
            [pltpu.VMEM((128, D), jnp.float32) for _ in range(NBUF)],
            [pltpu.VMEM((D * 128,), jnp.float32) for _ in range(NBUF)],
            [pltpu.SemaphoreType.DMA for _ in range(NBUF)],
            [pltpu.SemaphoreType.DMA for _ in range(NBUF)],
            pltpu.SemaphoreType.DMA,
        ],
    )(ids_lmajor, original_weight, new_weight)
    out5d = out5d.reshape(L, 8, BB, 8, 128)
    return out5d.transpose(2, 4, 0, 1, 3).reshape(B, L, D)


# speedup vs baseline: 1.7926x; 1.2789x over previous
import functools

import jax
import jax.numpy as jnp
from jax import lax
from jax.experimental import pallas as pl
from jax.experimental.pallas import tpu as pltpu
from jax.experimental.pallas import tpu_sc as plsc

N_ORIG = 1000000
N_NEW = 1024
D = 64
B = 16384
L = 50

NC = 2
NS = 16
NW = NC * NS

TOKENS = B * L
TPW = TOKENS // NW
CHUNK = 512
NCHUNK = TPW // CHUNK
SUB = 128
NSUB = CHUNK // SUB


def _body(ids_hbm, orig_hbm, new_hbm, out_hbm,
          idx0, idx1, oidx0, oidx1, rows0, rows1,
          gsem0, gsem1, wsem0, wsem1, psem):
    cid = lax.axis_index("c")
    sid = lax.axis_index("s")
    wid = sid * NC + cid
    base = wid * TPW

    idxb = (idx0, idx1)
    oidxb = (oidx0, oidx1)
    rowsb = (rows0, rows1)
    gsems = (gsem0, gsem1)
    wsems = (wsem0, wsem1)

    def load_fire(c, j):
        cb = base + c * CHUNK
        pltpu.sync_copy(ids_hbm.at[pl.ds(cb, CHUNK)], idxb[j])

        @pl.loop(0, CHUNK // 16)
        def _grp(g):
            v = idxb[j][pl.ds(g * 16, 16)]
            oidxb[j][pl.ds(g * 16, 16)] = jnp.where(v >= N_ORIG, 0, v)

        for s in range(NSUB):
            pltpu.async_copy(
                orig_hbm.at[oidxb[j].at[pl.ds(s * SUB, SUB)]],
                rowsb[j].at[pl.ds(s * SUB, SUB)],
                gsems[j],
            )

    def drain_gathers(j):
        for s in range(NSUB):
            pltpu.make_async_copy(
                orig_hbm.at[oidxb[j].at[pl.ds(s * SUB, SUB)]],
                rowsb[j].at[pl.ds(s * SUB, SUB)],
                gsems[j],
            ).wait()

    def fire_writeout(c, j):
        cb = base + c * CHUNK
        pltpu.async_copy(rowsb[j], out_hbm.at[pl.ds(cb, CHUNK)], wsems[j])

    def drain_writeout(c, j):
        cb = base + c * CHUNK
        pltpu.make_async_copy(
            rowsb[j], out_hbm.at[pl.ds(cb, CHUNK)], wsems[j]).wait()

    def patch(c, j):
        cb = base + c * CHUNK

        @pl.loop(0, CHUNK // 16)
        def _fix(g):
            v = idxb[j][pl.ds(g * 16, 16)]
            m = v >= N_ORIG
            cnt = plsc.all_reduce_population_count(m)[0]

            @pl.when(cnt > 0)
            def _():
                for t in range(16):
                    vid = v[t]

                    @pl.when(vid >= N_ORIG)
                    def _():
                        pltpu.async_copy(
                            new_hbm.at[pl.ds(vid - N_ORIG, 1)],
                            out_hbm.at[pl.ds(cb + g * 16 + t, 1)],
                            psem,
                        ).wait()

    load_fire(0, 0)

    @pl.loop(0, NCHUNK // 2)
    def _pair(p):
        for jj in range(2):
            c = 2 * p + jj
            prev = c - 1
            nxt = c + 1
            oj = 1 - jj

            if jj == 0:
                @pl.when(p > 0)
                def _():
                    drain_writeout(prev, oj)
                    patch(prev, oj)
                load_fire(nxt, oj)
            else:
                drain_writeout(prev, oj)
                patch(prev, oj)

                @pl.when(p < NCHUNK // 2 - 1)
                def _():
                    load_fire(nxt, oj)

            drain_gathers(jj)
            fire_writeout(c, jj)

    drain_writeout(NCHUNK - 1, 1)
    patch(NCHUNK - 1, 1)


@functools.partial(jax.jit, static_argnames=())
def kernel(input_ids, original_weight, new_weight):
    ids_flat = input_ids.reshape(TOKENS)
    mesh = plsc.VectorSubcoreMesh(
        core_axis_name="c", subcore_axis_name="s",
        num_cores=NC, num_subcores=NS,
    )
    out = pl.kernel(
        _body,
        out_type=jax.ShapeDtypeStruct((TOKENS, D), jnp.float32),
        mesh=mesh,
        compiler_params=pltpu.CompilerParams(
            use_tc_tiling_on_sc=False, needs_layout_passes=False),
        scratch_types=[
            pltpu.VMEM((CHUNK,), jnp.int32),
            pltpu.VMEM((CHUNK,), jnp.int32),
            pltpu.VMEM((CHUNK,), jnp.int32),
            pltpu.VMEM((CHUNK,), jnp.int32),
            pltpu.VMEM((CHUNK, D), jnp.float32),
            pltpu.VMEM((CHUNK, D), jnp.float32),
            pltpu.SemaphoreType.DMA,
            pltpu.SemaphoreType.DMA,
            pltpu.SemaphoreType.DMA,
            pltpu.SemaphoreType.DMA,
            pltpu.SemaphoreType.DMA,
        ],
    )(ids_flat, original_weight, new_weight)
    return out.reshape(B, L, D)
